# baseline (speedup 1.0000x reference)
import functools

import jax
import jax.numpy as jnp
from jax import lax
from jax.experimental import pallas as pl
from jax.experimental.pallas import tpu as pltpu
from jax.experimental.pallas import tpu_sc as plsc

_L = 16
_D = 128
_K = 22
_XB, _WT, _B = 0, _K * _L, 2 * _K * _L
_PACK = 2 * _K * _L + _L


def _hero_body(pack_hbm, idx_hbm, prim_hbm, atk_hbm, roles_hbm, out_hbm,
               pack_v, idx_v, prim_v, atk_v, roles_v, obuf_v,
               sem_par, sem_tab, sem_out):
    is_w0 = jnp.logical_and(lax.axis_index("c") == 0, lax.axis_index("s") == 0)

    @pl.when(is_w0)
    def _():
        cp_p = pltpu.async_copy(pack_hbm, pack_v, sem_par)
        cp_i = pltpu.async_copy(idx_hbm, idx_v, sem_par)
        cp_t0 = pltpu.async_copy(prim_hbm, prim_v, sem_tab)
        cp_t1 = pltpu.async_copy(atk_hbm, atk_v, sem_tab)
        cp_t2 = pltpu.async_copy(roles_hbm, roles_v, sem_tab)
        cp_p.wait()

        y = pack_v[pl.ds(_B, _L)]
        for k in range(_K):
            y = (y + pack_v[pl.ds(_WT + k * _L, _L)]
                 * pack_v[pl.ds(_XB + k * _L, _L)])
        obuf_v[pl.ds(3 * _D, _L)] = y

        cp_i.wait()
        idx = idx_v[...]

        dnums = lax.GatherDimensionNumbers(
            offset_dims=(), collapsed_slice_dims=(0,), start_index_map=(0,))

        def bcast(lane):
            starts = jnp.full((_L, 1), lane, jnp.int32)
            return lax.gather(idx, starts, dnums, (1,),
                              mode=lax.GatherScatterMode.PROMISE_IN_BOUNDS)

        one = jnp.full((_L,), 1.0, jnp.float32)
        zero = jnp.zeros((_L,), jnp.float32)
        fifth = jnp.full((_L,), 0.2, jnp.float32)
        bp = bcast(0)
        ba = bcast(1)
        br = [bcast(2 + j) for j in range(5)]
        wp = [jnp.where(bp == v, one, zero) for v in range(4)]
        wa = [jnp.where(ba == v, one, zero) for v in range(2)]
        wr = []
        for v in range(8):
            acc = zero
            for j in range(5):
                acc = acc + jnp.where(br[j] == v, fifth, zero)
            wr.append(acc)

        cp_t0.wait()
        cp_t1.wait()
        cp_t2.wait()
        for c in range(_D // _L):
            s = pl.ds(c * _L, _L)
            accp = wp[0] * prim_v[0, s]
            for v in range(1, 4):
                accp = accp + wp[v] * prim_v[v, s]
            obuf_v[pl.ds(0 * _D + c * _L, _L)] = accp
            obuf_v[pl.ds(1 * _D + c * _L, _L)] = (
                wa[0] * atk_v[0, s] + wa[1] * atk_v[1, s])
            accr = wr[0] * roles_v[0, s]
            for v in range(1, 8):
                accr = accr + wr[v] * roles_v[v, s]
            obuf_v[pl.ds(2 * _D + c * _L, _L)] = accr
        pltpu.async_copy(obuf_v, out_hbm, sem_out).wait()


_hero_sc = functools.partial(
    pl.kernel,
    out_type=jax.ShapeDtypeStruct((3 * _D + _L,), jnp.float32),
    mesh=plsc.VectorSubcoreMesh(core_axis_name="c", subcore_axis_name="s",
                                num_cores=1, num_subcores=1),
    scratch_types=[
        pltpu.VMEM((_PACK,), jnp.float32),
        pltpu.VMEM((_L,), jnp.int32),
        pltpu.VMEM((4, _D), jnp.float32),
        pltpu.VMEM((2, _D), jnp.float32),
        pltpu.VMEM((8, _D), jnp.float32),
        pltpu.VMEM((3 * _D + _L,), jnp.float32),
        pltpu.SemaphoreType.DMA,
        pltpu.SemaphoreType.DMA,
        pltpu.SemaphoreType.DMA,
    ],
)(_hero_body)


def kernel(p_attrs, a_types, role_i, float_stats, primary_table,
           attack_table, roles_table, proj_W, proj_b):
    idx16 = jnp.concatenate([
        p_attrs.astype(jnp.int32),
        a_types.astype(jnp.int32),
        role_i.astype(jnp.int32),
        jnp.zeros((_L - 7,), jnp.int32),
    ])
    pack = jnp.concatenate([
        jnp.repeat(float_stats, _L),
        proj_W.T.reshape(-1),
        proj_b,
    ])
    return _hero_sc(pack, idx16, primary_table, attack_table, roles_table)


# speedup vs baseline: 1.0645x; 1.0645x over previous
import functools

import jax
import jax.numpy as jnp
from jax import lax
from jax.experimental import pallas as pl
from jax.experimental.pallas import tpu as pltpu
from jax.experimental.pallas import tpu_sc as plsc

_L = 16
_D = 128
_K = 22

_DNUMS = lax.GatherDimensionNumbers(
    offset_dims=(), collapsed_slice_dims=(0,), start_index_map=(0,))


def _dyn_gather(vec, idx):
    return lax.gather(vec, idx[:, None], _DNUMS, (1,),
                      mode=lax.GatherScatterMode.PROMISE_IN_BOUNDS)


def _bcast_lane(vec, lane):
    return _dyn_gather(vec, jnp.full((_L,), lane, jnp.int32))


def _hero_body(p_hbm, a_hbm, r_hbm, x_hbm, prim_hbm, atk_hbm, roles_hbm,
               w_hbm, b_hbm, out_hbm,
               pidx_v, aidx_v, ridx_v, x_v, w_v, b_v,
               prim_v, atk_v, roles_v, obuf_v,
               sem_idx, sem_par, sem_tab, sem_out):
    is_w0 = jnp.logical_and(lax.axis_index("c") == 0, lax.axis_index("s") == 0)

    @pl.when(is_w0)
    def _():
        cp_p = pltpu.async_copy(p_hbm, pidx_v.at[pl.ds(0, 1)], sem_idx)
        cp_a = pltpu.async_copy(a_hbm, aidx_v.at[pl.ds(0, 1)], sem_idx)
        cp_r = pltpu.async_copy(r_hbm, ridx_v.at[pl.ds(0, 5)], sem_idx)
        cp_x = pltpu.async_copy(x_hbm, x_v, sem_par)
        cp_w = pltpu.async_copy(w_hbm, w_v, sem_par)
        cp_b = pltpu.async_copy(b_hbm, b_v, sem_par)
        cp_t0 = pltpu.async_copy(prim_hbm, prim_v, sem_tab)
        cp_t1 = pltpu.async_copy(atk_hbm, atk_v, sem_tab)
        cp_t2 = pltpu.async_copy(roles_hbm, roles_v, sem_tab)

        cp_x.wait()
        cp_w.wait()
        cp_b.wait()
        x0 = x_v[pl.ds(0, _L)]
        x1 = x_v[pl.ds(_K - _L, _L)]
        lane = lax.iota(jnp.int32, _L)
        zero = jnp.zeros((_L,), jnp.float32)
        y = b_v[...]
        for j in range(_L):
            t = (w_v[j, pl.ds(0, _L)] * x0
                 + jnp.where(lane < 2 * _L - _K, zero,
                             w_v[j, pl.ds(_K - _L, _L)] * x1))
            for m in (8, 4, 2, 1):
                t = t + _dyn_gather(t, lane ^ m)
            y = y + jnp.where(lane == j, t, zero)
        obuf_v[pl.ds(3 * _D, _L)] = y

        cp_p.wait()
        cp_a.wait()
        cp_r.wait()
        bp = _bcast_lane(pidx_v[...], 0)
        ba = _bcast_lane(aidx_v[...], 0)
        ridx = ridx_v[...]
        br = [_bcast_lane(ridx, j) for j in range(5)]
        one = jnp.full((_L,), 1.0, jnp.float32)
        fifth = jnp.full((_L,), 0.2, jnp.float32)
        wp = [jnp.where(bp == v, one, zero) for v in range(4)]
        wa = [jnp.where(ba == v, one, zero) for v in range(2)]
        wr = []
        for v in range(8):
            acc = zero
            for j in range(5):
                acc = acc + jnp.where(br[j] == v, fifth, zero)
            wr.append(acc)

        cp_t0.wait()
        cp_t1.wait()
        cp_t2.wait()
        for c in range(_D // _L):
            s = pl.ds(c * _L, _L)
            accp = wp[0] * prim_v[0, s]
            for v in range(1, 4):
                accp = accp + wp[v] * prim_v[v, s]
            obuf_v[pl.ds(0 * _D + c * _L, _L)] = accp
            obuf_v[pl.ds(1 * _D + c * _L, _L)] = (
                wa[0] * atk_v[0, s] + wa[1] * atk_v[1, s])
            accr = wr[0] * roles_v[0, s]
            for v in range(1, 8):
                accr = accr + wr[v] * roles_v[v, s]
            obuf_v[pl.ds(2 * _D + c * _L, _L)] = accr
        pltpu.async_copy(obuf_v, out_hbm, sem_out).wait()


_hero_sc = functools.partial(
    pl.kernel,
    out_type=jax.ShapeDtypeStruct((3 * _D + _L,), jnp.float32),
    mesh=plsc.VectorSubcoreMesh(core_axis_name="c", subcore_axis_name="s",
                                num_cores=1, num_subcores=1),
    scratch_types=[
        pltpu.VMEM((_L,), jnp.int32),
        pltpu.VMEM((_L,), jnp.int32),
        pltpu.VMEM((_L,), jnp.int32),
        pltpu.VMEM((_K,), jnp.float32),
        pltpu.VMEM((_L, _K), jnp.float32),
        pltpu.VMEM((_L,), jnp.float32),
        pltpu.VMEM((4, _D), jnp.float32),
        pltpu.VMEM((2, _D), jnp.float32),
        pltpu.VMEM((8, _D), jnp.float32),
        pltpu.VMEM((3 * _D + _L,), jnp.float32),
        pltpu.SemaphoreType.DMA,
        pltpu.SemaphoreType.DMA,
        pltpu.SemaphoreType.DMA,
        pltpu.SemaphoreType.DMA,
    ],
)(_hero_body)


def kernel(p_attrs, a_types, role_i, float_stats, primary_table,
           attack_table, roles_table, proj_W, proj_b):
    return _hero_sc(
        p_attrs.astype(jnp.int32),
        a_types.astype(jnp.int32),
        role_i.astype(jnp.int32),
        float_stats,
        primary_table,
        attack_table,
        roles_table,
        proj_W,
        proj_b,
    )


# speedup vs baseline: 1.0860x; 1.0202x over previous
import functools

import jax
import jax.numpy as jnp
from jax import lax
from jax.experimental import pallas as pl
from jax.experimental.pallas import tpu as pltpu
from jax.experimental.pallas import tpu_sc as plsc

_L = 16
_D = 128
_K = 22

_DNUMS = lax.GatherDimensionNumbers(
    offset_dims=(), collapsed_slice_dims=(0,), start_index_map=(0,))


def _dyn_gather(vec, idx):
    return lax.gather(vec, idx[:, None], _DNUMS, (1,),
                      mode=lax.GatherScatterMode.PROMISE_IN_BOUNDS)


def _bcast_lane(vec, lane):
    return _dyn_gather(vec, jnp.full((_L,), lane, jnp.int32))


def _hero_body(p_hbm, a_hbm, r_hbm, x_hbm, prim_hbm, atk_hbm, roles_hbm,
               w_hbm, b_hbm, out_hbm,
               pidx_v, aidx_v, ridx_v, x_v, w_v, b_v,
               prim_v, atk_v, roles_v, obuf_v, y_v,
               sem_idx, sem_par, sem_tab, sem_out):
    on_core0 = lax.axis_index("c") == 0
    sid = lax.axis_index("s")
    lane = lax.iota(jnp.int32, _L)
    zero = jnp.zeros((_L,), jnp.float32)

    @pl.when(jnp.logical_and(on_core0, sid == 0))
    def _embed():
        cp_p = pltpu.async_copy(p_hbm, pidx_v.at[pl.ds(0, 1)], sem_idx)
        cp_a = pltpu.async_copy(a_hbm, aidx_v.at[pl.ds(0, 1)], sem_idx)
        cp_r = pltpu.async_copy(r_hbm, ridx_v.at[pl.ds(0, 5)], sem_idx)
        cp_t0 = pltpu.async_copy(prim_hbm, prim_v, sem_tab)
        cp_t1 = pltpu.async_copy(atk_hbm, atk_v, sem_tab)
        cp_t2 = pltpu.async_copy(roles_hbm, roles_v, sem_tab)

        cp_p.wait()
        cp_a.wait()
        cp_r.wait()
        bp = _bcast_lane(pidx_v[...], 0)
        ba = _bcast_lane(aidx_v[...], 0)
        ridx = ridx_v[...]
        br = [_bcast_lane(ridx, j) for j in range(5)]
        one = jnp.full((_L,), 1.0, jnp.float32)
        fifth = jnp.full((_L,), 0.2, jnp.float32)
        wp = [jnp.where(bp == v, one, zero) for v in range(4)]
        wa = [jnp.where(ba == v, one, zero) for v in range(2)]
        wr = []
        for v in range(8):
            acc = zero
            for j in range(5):
                acc = acc + jnp.where(br[j] == v, fifth, zero)
            wr.append(acc)

        cp_t0.wait()
        cp_t1.wait()
        cp_t2.wait()
        for c in range(_D // _L):
            s = pl.ds(c * _L, _L)
            accp = wp[0] * prim_v[0, s]
            for v in range(1, 4):
                accp = accp + wp[v] * prim_v[v, s]
            obuf_v[pl.ds(0 * _D + c * _L, _L)] = accp
            obuf_v[pl.ds(1 * _D + c * _L, _L)] = (
                wa[0] * atk_v[0, s] + wa[1] * atk_v[1, s])
            accr = wr[0] * roles_v[0, s]
            for v in range(1, 8):
                accr = accr + wr[v] * roles_v[v, s]
            obuf_v[pl.ds(2 * _D + c * _L, _L)] = accr
        pltpu.async_copy(obuf_v, out_hbm.at[pl.ds(0, 3 * _D)], sem_out).wait()

    @pl.when(jnp.logical_and(on_core0, sid == 1))
    def _linear():
        cp_x = pltpu.async_copy(x_hbm, x_v, sem_par)
        cp_w = pltpu.async_copy(w_hbm, w_v, sem_par)
        cp_b = pltpu.async_copy(b_hbm, b_v, sem_par)

        cp_x.wait()
        cp_w.wait()
        cp_b.wait()
        x0 = x_v[pl.ds(0, _L)]
        x1 = x_v[pl.ds(_K - _L, _L)]
        y = b_v[...]
        for j in range(_L):
            t = (w_v[j, pl.ds(0, _L)] * x0
                 + jnp.where(lane < 2 * _L - _K, zero,
                             w_v[j, pl.ds(_K - _L, _L)] * x1))
            for m in (8, 4, 2, 1):
                t = t + _dyn_gather(t, lane ^ m)
            y = y + jnp.where(lane == j, t, zero)
        y_v[...] = y
        pltpu.async_copy(y_v, out_hbm.at[pl.ds(3 * _D, _L)], sem_out).wait()


_hero_sc = functools.partial(
    pl.kernel,
    out_type=jax.ShapeDtypeStruct((3 * _D + _L,), jnp.float32),
    mesh=plsc.VectorSubcoreMesh(core_axis_name="c", subcore_axis_name="s",
                                num_cores=1, num_subcores=2),
    scratch_types=[
        pltpu.VMEM((_L,), jnp.int32),
        pltpu.VMEM((_L,), jnp.int32),
        pltpu.VMEM((_L,), jnp.int32),
        pltpu.VMEM((_K,), jnp.float32),
        pltpu.VMEM((_L, _K), jnp.float32),
        pltpu.VMEM((_L,), jnp.float32),
        pltpu.VMEM((4, _D), jnp.float32),
        pltpu.VMEM((2, _D), jnp.float32),
        pltpu.VMEM((8, _D), jnp.float32),
        pltpu.VMEM((3 * _D,), jnp.float32),
        pltpu.VMEM((_L,), jnp.float32),
        pltpu.SemaphoreType.DMA,
        pltpu.SemaphoreType.DMA,
        pltpu.SemaphoreType.DMA,
        pltpu.SemaphoreType.DMA,
    ],
)(_hero_body)


def kernel(p_attrs, a_types, role_i, float_stats, primary_table,
           attack_table, roles_table, proj_W, proj_b):
    return _hero_sc(
        p_attrs.astype(jnp.int32),
        a_types.astype(jnp.int32),
        role_i.astype(jnp.int32),
        float_stats,
        primary_table,
        attack_table,
        roles_table,
        proj_W,
        proj_b,
    )
